# Initial kernel scaffold; baseline (speedup 1.0000x reference)
import functools

import jax
import jax.numpy as jnp
from jax import lax
from jax.experimental import pallas as pl
from jax.experimental.pallas import tpu as pltpu
from jax.experimental.pallas import tpu_sc as plsc

N = 100000
E = 1600000
H = 32
HH = 16
G = 64
NCLS = 8

CH = 128
SUPR = 16
NSUP = 49
RPS = SUPR * NSUP
E_PAD = 16 * RPS * CH
NROW = E_PAD // CH
OUTR = 6272
NPAD = 16 * OUTR
DSUPR = 8
DNSUP = 49
DRPW = DSUPR * DNSUP

BN = 1000
GRID = N // BN

_mesh = plsc.VectorSubcoreMesh(core_axis_name="c", subcore_axis_name="s")
_f32 = jnp.float32
_acc_sh = jax.ShapeDtypeStruct((NPAD, HH), _f32)


def _fill(buf, val):
    @pl.loop(0, buf.shape[0])
    def _(i):
        buf[i, :] = jnp.full((HH,), val, _f32)


def _zero_acc(acc, zbuf, s):
    _fill(zbuf, 0.0)
    zbase = s * OUTR

    @pl.loop(0, OUTR // CH)
    def _(j):
        pltpu.sync_copy(zbuf, acc.at[pl.ds(zbase + j * CH, CH)])


def _copy_out(acc, c, s, o0_hbm, o1_hbm):
    ob = s * OUTR

    @pl.when(c == 0)
    def _():
        pltpu.sync_copy(acc.at[pl.ds(ob, OUTR)], o0_hbm.at[pl.ds(ob, OUTR)])

    @pl.when(c == 1)
    def _():
        pltpu.sync_copy(acc.at[pl.ds(ob, OUTR)], o1_hbm.at[pl.ds(ob, OUTR)])


@jax.jit
def _sc_deg(dst2d):
    @functools.partial(
        pl.kernel,
        out_type=(_acc_sh, _acc_sh),
        mesh=_mesh,
        scratch_types=[
            pltpu.VMEM_SHARED((NPAD, HH), _f32),
            pltpu.VMEM((DSUPR, CH), jnp.int32),
            pltpu.VMEM((CH, HH), _f32),
            pltpu.VMEM((CH, HH), _f32),
        ],
    )
    def k(dst_hbm, o0_hbm, o1_hbm, acc, didx, ones, zbuf):
        c = lax.axis_index("c")
        s = lax.axis_index("s")
        _zero_acc(acc, zbuf, s)
        _fill(ones, 1.0)
        plsc.subcore_barrier()

        w = s * 2 + c
        rbase = w * DRPW

        @pl.loop(0, DNSUP)
        def _(q):
            pltpu.sync_copy(dst_hbm.at[pl.ds(rbase + q * DSUPR, DSUPR)], didx)

            @pl.loop(0, DSUPR)
            def _(kk):
                pltpu.sync_copy(ones, acc.at[didx.at[kk]], add=True)

        plsc.subcore_barrier()
        _copy_out(acc, c, s, o0_hbm, o1_hbm)

    return k(dst2d)


@jax.jit
def _sc_scatter(g0, g1, src2d, dst2d):
    @functools.partial(
        pl.kernel,
        out_type=(_acc_sh, _acc_sh),
        mesh=_mesh,
        scratch_types=[
            pltpu.VMEM_SHARED((NPAD, HH), _f32),
            pltpu.VMEM((SUPR, CH), jnp.int32),
            pltpu.VMEM((SUPR, CH), jnp.int32),
            pltpu.VMEM((CH, HH), _f32),
            pltpu.VMEM((CH, HH), _f32),
        ],
    )
    def k(g0_hbm, g1_hbm, src_hbm, dst_hbm, o0_hbm, o1_hbm,
          acc, sidx, didx, rows, zbuf):
        c = lax.axis_index("c")
        s = lax.axis_index("s")
        _zero_acc(acc, zbuf, s)
        plsc.subcore_barrier()

        rbase = s * RPS

        def run(g_hbm):
            @pl.loop(0, NSUP)
            def _(q):
                r0 = rbase + q * SUPR
                pltpu.sync_copy(src_hbm.at[pl.ds(r0, SUPR)], sidx)
                pltpu.sync_copy(dst_hbm.at[pl.ds(r0, SUPR)], didx)

                @pl.loop(0, SUPR)
                def _(kk):
                    pltpu.sync_copy(g_hbm.at[sidx.at[kk]], rows)
                    pltpu.sync_copy(rows, acc.at[didx.at[kk]], add=True)

        @pl.when(c == 0)
        def _():
            run(g0_hbm)

        @pl.when(c == 1)
        def _():
            run(g1_hbm)

        plsc.subcore_barrier()
        _copy_out(acc, c, s, o0_hbm, o1_hbm)

    return k(g0, g1, src2d, dst2d)


def _full_spec(shape):
    return pl.BlockSpec(shape, lambda i: tuple(0 for _ in shape))


def _row_spec(w):
    return pl.BlockSpec((BN, w), lambda i: (i, 0))


@jax.jit
def _tc_init(nt2, xnum, d0, d1, W1, b1r, W2, b2r, Wg0):
    def body(nt_ref, xn_ref, d0_ref, d1_ref, W1_ref, b1_ref, W2_ref, b2_ref,
             Wg_ref, g0_ref, g1_ref, dv_ref):
        nt = nt_ref[...]
        oh = (nt == lax.broadcasted_iota(jnp.int32, (BN, NCLS), 1)
              ).astype(_f32)
        xt = jnp.dot(oh, W1_ref[...], preferred_element_type=_f32) + b1_ref[...]
        xn = jnp.dot(xn_ref[...], W2_ref[...], preferred_element_type=_f32) + b2_ref[...]
        deg = d0_ref[...][:, 0:1] + d1_ref[...][:, 0:1] + 1.0
        dinv = lax.rsqrt(deg)
        Wg = Wg_ref[...]
        h = (jnp.dot(xt, Wg[:H, :], preferred_element_type=_f32)
             + jnp.dot(xn, Wg[H:, :], preferred_element_type=_f32))
        g = h * dinv
        g0_ref[...] = g[:, :HH]
        g1_ref[...] = g[:, HH:]
        dv_ref[...] = dinv

    return pl.pallas_call(
        body,
        grid=(GRID,),
        in_specs=[
            _row_spec(1), _row_spec(4), _row_spec(HH), _row_spec(HH),
            _full_spec((NCLS, H)), _full_spec((1, H)),
            _full_spec((4, H)), _full_spec((1, H)),
            _full_spec((2 * H, H)),
        ],
        out_specs=[_row_spec(HH), _row_spec(HH), _row_spec(1)],
        out_shape=[
            jax.ShapeDtypeStruct((N, HH), _f32),
            jax.ShapeDtypeStruct((N, HH), _f32),
            jax.ShapeDtypeStruct((N, 1), _f32),
        ],
    )(nt2, xnum, d0, d1, W1, b1r, W2, b2r, Wg0)


@jax.jit
def _tc_layer(s0, s1, g0, g1, dv, br, W):
    def body(s0_ref, s1_ref, g0_ref, g1_ref, dv_ref, b_ref, W_ref,
             o0_ref, o1_ref):
        zl = s0_ref[...] + g0_ref[...]
        zr = s1_ref[...] + g1_ref[...]
        dinv = dv_ref[...]
        z = jnp.concatenate([zl, zr], axis=1) * dinv + b_ref[...]
        z = jnp.maximum(z, 0.0)
        h = jnp.dot(z, W_ref[...], preferred_element_type=_f32)
        g = h * dinv
        o0_ref[...] = g[:, :HH]
        o1_ref[...] = g[:, HH:]

    return pl.pallas_call(
        body,
        grid=(GRID,),
        in_specs=[
            _row_spec(HH), _row_spec(HH), _row_spec(HH), _row_spec(HH),
            _row_spec(1), _full_spec((1, H)), _full_spec((H, H)),
        ],
        out_specs=[_row_spec(HH), _row_spec(HH)],
        out_shape=[
            jax.ShapeDtypeStruct((N, HH), _f32),
            jax.ShapeDtypeStruct((N, HH), _f32),
        ],
    )(s0, s1, g0, g1, dv, br, W)


@jax.jit
def _tc_final(s0, s1, g0, g1, dv, br, bt2, Wout, boutr):
    def body(s0_ref, s1_ref, g0_ref, g1_ref, dv_ref, b_ref, bt_ref,
             Wo_ref, bo_ref, out_ref, sums, cnts):
        i = pl.program_id(0)

        @pl.when(i == 0)
        def _():
            sums[...] = jnp.zeros((G, H), _f32)
            cnts[...] = jnp.zeros((G, 1), _f32)

        zl = s0_ref[...] + g0_ref[...]
        zr = s1_ref[...] + g1_ref[...]
        z = jnp.concatenate([zl, zr], axis=1) * dv_ref[...] + b_ref[...]
        bt = bt_ref[...]
        oh = (bt == lax.broadcasted_iota(jnp.int32, (BN, G), 1)).astype(_f32)
        dn = (((0,), (0,)), ((), ()))
        sums[...] += lax.dot_general(oh, z, dn, preferred_element_type=_f32)
        cnts[...] += lax.dot_general(oh, jnp.ones((BN, 1), _f32), dn,
                                     preferred_element_type=_f32)

        @pl.when(i == GRID - 1)
        def _():
            mean = sums[...] / jnp.maximum(cnts[...], 1.0)
            out_ref[...] = (jnp.dot(mean, Wo_ref[...],
                                    preferred_element_type=_f32) + bo_ref[...])

    return pl.pallas_call(
        body,
        grid=(GRID,),
        in_specs=[
            _row_spec(HH), _row_spec(HH), _row_spec(HH), _row_spec(HH),
            _row_spec(1), _full_spec((1, H)), _row_spec(1),
            _full_spec((H, 1)), _full_spec((1, 1)),
        ],
        out_specs=[pl.BlockSpec((G, 1), lambda i: (0, 0))],
        out_shape=jax.ShapeDtypeStruct((G, 1), _f32),
        scratch_shapes=[
            pltpu.VMEM((G, H), _f32),
            pltpu.VMEM((G, 1), _f32),
        ],
    )(s0, s1, g0, g1, dv, br, bt2, Wout, boutr)


def kernel(node_type, c, gm, pos, r, edge_index, batch, W1, b1, W2, b2,
           Wg0, bg0, Wg1, bg1, Wg2, bg2, Wg3, bg3, Wg4, bg4, Wout, bout):
    src = edge_index[0].astype(jnp.int32)
    dst = edge_index[1].astype(jnp.int32)
    pad = E_PAD - E
    src2d = jnp.concatenate(
        [src, jnp.zeros((pad,), jnp.int32)]).reshape(NROW, CH)
    dst2d = jnp.concatenate(
        [dst, jnp.full((pad,), N, jnp.int32)]).reshape(NROW, CH)
    xnum = jnp.stack([c, gm, pos, r], axis=-1)
    nt2 = node_type.astype(jnp.int32).reshape(N, 1)
    bt2 = batch.astype(jnp.int32).reshape(N, 1)

    d0, d1 = _sc_deg(dst2d)
    g0, g1, dv = _tc_init(nt2, xnum, d0, d1, W1, b1.reshape(1, H),
                          W2, b2.reshape(1, H), Wg0)
    Ws = [Wg1, Wg2, Wg3, Wg4]
    bs = [bg0, bg1, bg2, bg3]
    for i in range(4):
        s0, s1 = _sc_scatter(g0, g1, src2d, dst2d)
        g0, g1 = _tc_layer(s0, s1, g0, g1, dv, bs[i].reshape(1, H), Ws[i])
    s0, s1 = _sc_scatter(g0, g1, src2d, dst2d)
    return _tc_final(s0, s1, g0, g1, dv, bg4.reshape(1, H), bt2,
                     Wout, bout.reshape(1, 1))


# speedup vs baseline: 10.8119x; 10.8119x over previous
import functools

import jax
import jax.numpy as jnp
from jax import lax
from jax.experimental import pallas as pl
from jax.experimental.pallas import tpu as pltpu
from jax.experimental.pallas import tpu_sc as plsc

N = 100000
E = 1600000
H = 32
HH = 16
G = 64
NCLS = 8

CH = 128
SUPR = 16
NSUP = 49
RPS = SUPR * NSUP
E_PAD = 16 * RPS * CH
NROW = E_PAD // CH
OUTR = 6256
OUTR_LAST = N - 15 * OUTR
ACCR = 16 * OUTR
DSUPR = 8
DNSUP = 49
DRPW = DSUPR * DNSUP

BN = 1000
GRID = N // BN

_mesh = plsc.VectorSubcoreMesh(core_axis_name="c", subcore_axis_name="s")
_sc_params = pltpu.CompilerParams(use_tc_tiling_on_sc=False)
_f32 = jnp.float32
_out_sh = jax.ShapeDtypeStruct((N, HH), _f32)


def _zero_acc(acc, zeros_hbm, s):
    pltpu.sync_copy(zeros_hbm, acc.at[pl.ds(s * OUTR, OUTR)])


def _copy_out(acc, c, s, o0_hbm, o1_hbm):
    ob = s * OUTR

    def emit(o_hbm, nrows):
        pltpu.sync_copy(acc.at[pl.ds(ob, nrows)], o_hbm.at[pl.ds(ob, nrows)])

    for core, o_hbm in ((0, o0_hbm), (1, o1_hbm)):
        @pl.when(jnp.logical_and(c == core, s < 15))
        def _():
            emit(o_hbm, OUTR)

        @pl.when(jnp.logical_and(c == core, s == 15))
        def _():
            emit(o_hbm, OUTR_LAST)


@jax.jit
def _sc_deg(dst2d, zeros_hbm, ones_hbm):
    @functools.partial(
        pl.kernel,
        out_type=(_out_sh, _out_sh),
        mesh=_mesh,
        compiler_params=_sc_params,
        scratch_types=[
            pltpu.VMEM_SHARED((ACCR, HH), _f32),
            pltpu.VMEM((DSUPR, CH), jnp.int32),
            pltpu.VMEM((CH, HH), _f32),
        ],
    )
    def k(dst_hbm, z_hbm, one_hbm, o0_hbm, o1_hbm, acc, didx, ones):
        c = lax.axis_index("c")
        s = lax.axis_index("s")
        _zero_acc(acc, z_hbm, s)
        pltpu.sync_copy(one_hbm, ones)
        plsc.subcore_barrier()

        w = s * 2 + c
        rbase = w * DRPW

        @pl.loop(0, DNSUP)
        def _(q):
            pltpu.sync_copy(dst_hbm.at[pl.ds(rbase + q * DSUPR, DSUPR)], didx)

            @pl.loop(0, DSUPR)
            def _(kk):
                pltpu.sync_copy(ones, acc.at[didx.at[kk]], add=True)

        plsc.subcore_barrier()
        _copy_out(acc, c, s, o0_hbm, o1_hbm)

    return k(dst2d, zeros_hbm, ones_hbm)


@jax.jit
def _sc_scatter(g0, g1, src2d, dst2d, zeros_hbm):
    @functools.partial(
        pl.kernel,
        out_type=(_out_sh, _out_sh),
        mesh=_mesh,
        compiler_params=_sc_params,
        scratch_types=[
            pltpu.VMEM_SHARED((ACCR, HH), _f32),
            pltpu.VMEM((SUPR, CH), jnp.int32),
            pltpu.VMEM((SUPR, CH), jnp.int32),
            pltpu.VMEM((CH, HH), _f32),
        ],
    )
    def k(g0_hbm, g1_hbm, src_hbm, dst_hbm, z_hbm, o0_hbm, o1_hbm,
          acc, sidx, didx, rows):
        c = lax.axis_index("c")
        s = lax.axis_index("s")
        _zero_acc(acc, z_hbm, s)
        plsc.subcore_barrier()

        rbase = s * RPS

        def run(g_hbm):
            @pl.loop(0, NSUP)
            def _(q):
                r0 = rbase + q * SUPR
                pltpu.sync_copy(src_hbm.at[pl.ds(r0, SUPR)], sidx)
                pltpu.sync_copy(dst_hbm.at[pl.ds(r0, SUPR)], didx)

                @pl.loop(0, SUPR)
                def _(kk):
                    pltpu.sync_copy(g_hbm.at[sidx.at[kk]], rows)
                    pltpu.sync_copy(rows, acc.at[didx.at[kk]], add=True)

        @pl.when(c == 0)
        def _():
            run(g0_hbm)

        @pl.when(c == 1)
        def _():
            run(g1_hbm)

        plsc.subcore_barrier()
        _copy_out(acc, c, s, o0_hbm, o1_hbm)

    return k(g0, g1, src2d, dst2d, zeros_hbm)


def _full_spec(shape):
    return pl.BlockSpec(shape, lambda i: tuple(0 for _ in shape))


def _row_spec(w):
    return pl.BlockSpec((BN, w), lambda i: (i, 0))


@jax.jit
def _tc_init(nt2, xnum, d0, d1, W1, b1r, W2, b2r, Wg0):
    def body(nt_ref, xn_ref, d0_ref, d1_ref, W1_ref, b1_ref, W2_ref, b2_ref,
             Wg_ref, g0_ref, g1_ref, dv_ref):
        nt = nt_ref[...]
        oh = (nt == lax.broadcasted_iota(jnp.int32, (BN, NCLS), 1)
              ).astype(_f32)
        xt = jnp.dot(oh, W1_ref[...], preferred_element_type=_f32) + b1_ref[...]
        xn = jnp.dot(xn_ref[...], W2_ref[...], preferred_element_type=_f32) + b2_ref[...]
        deg = d0_ref[...][:, 0:1] + d1_ref[...][:, 0:1] + 1.0
        dinv = lax.rsqrt(deg)
        Wg = Wg_ref[...]
        h = (jnp.dot(xt, Wg[:H, :], preferred_element_type=_f32)
             + jnp.dot(xn, Wg[H:, :], preferred_element_type=_f32))
        g = h * dinv
        g0_ref[...] = g[:, :HH]
        g1_ref[...] = g[:, HH:]
        dv_ref[...] = dinv

    return pl.pallas_call(
        body,
        grid=(GRID,),
        in_specs=[
            _row_spec(1), _row_spec(4), _row_spec(HH), _row_spec(HH),
            _full_spec((NCLS, H)), _full_spec((1, H)),
            _full_spec((4, H)), _full_spec((1, H)),
            _full_spec((2 * H, H)),
        ],
        out_specs=[_row_spec(HH), _row_spec(HH), _row_spec(1)],
        out_shape=[
            jax.ShapeDtypeStruct((N, HH), _f32),
            jax.ShapeDtypeStruct((N, HH), _f32),
            jax.ShapeDtypeStruct((N, 1), _f32),
        ],
    )(nt2, xnum, d0, d1, W1, b1r, W2, b2r, Wg0)


@jax.jit
def _tc_layer(s0, s1, g0, g1, dv, br, W):
    def body(s0_ref, s1_ref, g0_ref, g1_ref, dv_ref, b_ref, W_ref,
             o0_ref, o1_ref):
        zl = s0_ref[...] + g0_ref[...]
        zr = s1_ref[...] + g1_ref[...]
        dinv = dv_ref[...]
        z = jnp.concatenate([zl, zr], axis=1) * dinv + b_ref[...]
        z = jnp.maximum(z, 0.0)
        h = jnp.dot(z, W_ref[...], preferred_element_type=_f32)
        g = h * dinv
        o0_ref[...] = g[:, :HH]
        o1_ref[...] = g[:, HH:]

    return pl.pallas_call(
        body,
        grid=(GRID,),
        in_specs=[
            _row_spec(HH), _row_spec(HH), _row_spec(HH), _row_spec(HH),
            _row_spec(1), _full_spec((1, H)), _full_spec((H, H)),
        ],
        out_specs=[_row_spec(HH), _row_spec(HH)],
        out_shape=[
            jax.ShapeDtypeStruct((N, HH), _f32),
            jax.ShapeDtypeStruct((N, HH), _f32),
        ],
    )(s0, s1, g0, g1, dv, br, W)


@jax.jit
def _tc_final(s0, s1, g0, g1, dv, br, bt2, Wout, boutr):
    def body(s0_ref, s1_ref, g0_ref, g1_ref, dv_ref, b_ref, bt_ref,
             Wo_ref, bo_ref, out_ref, sums, cnts):
        i = pl.program_id(0)

        @pl.when(i == 0)
        def _():
            sums[...] = jnp.zeros((G, H), _f32)
            cnts[...] = jnp.zeros((G, 1), _f32)

        zl = s0_ref[...] + g0_ref[...]
        zr = s1_ref[...] + g1_ref[...]
        z = jnp.concatenate([zl, zr], axis=1) * dv_ref[...] + b_ref[...]
        bt = bt_ref[...]
        oh = (bt == lax.broadcasted_iota(jnp.int32, (BN, G), 1)).astype(_f32)
        dn = (((0,), (0,)), ((), ()))
        sums[...] += lax.dot_general(oh, z, dn, preferred_element_type=_f32)
        cnts[...] += lax.dot_general(oh, jnp.ones((BN, 1), _f32), dn,
                                     preferred_element_type=_f32)

        @pl.when(i == GRID - 1)
        def _():
            mean = sums[...] / jnp.maximum(cnts[...], 1.0)
            out_ref[...] = (jnp.dot(mean, Wo_ref[...],
                                    preferred_element_type=_f32) + bo_ref[...])

    return pl.pallas_call(
        body,
        grid=(GRID,),
        in_specs=[
            _row_spec(HH), _row_spec(HH), _row_spec(HH), _row_spec(HH),
            _row_spec(1), _full_spec((1, H)), _row_spec(1),
            _full_spec((H, 1)), _full_spec((1, 1)),
        ],
        out_specs=pl.BlockSpec((G, 1), lambda i: (0, 0)),
        out_shape=jax.ShapeDtypeStruct((G, 1), _f32),
        scratch_shapes=[
            pltpu.VMEM((G, H), _f32),
            pltpu.VMEM((G, 1), _f32),
        ],
    )(s0, s1, g0, g1, dv, br, bt2, Wout, boutr)


def kernel(node_type, c, gm, pos, r, edge_index, batch, W1, b1, W2, b2,
           Wg0, bg0, Wg1, bg1, Wg2, bg2, Wg3, bg3, Wg4, bg4, Wout, bout):
    src = edge_index[0].astype(jnp.int32)
    dst = edge_index[1].astype(jnp.int32)
    pad = E_PAD - E
    src2d = jnp.concatenate(
        [src, jnp.zeros((pad,), jnp.int32)]).reshape(NROW, CH)
    dst2d = jnp.concatenate(
        [dst, jnp.full((pad,), N, jnp.int32)]).reshape(NROW, CH)
    xnum = jnp.stack([c, gm, pos, r], axis=-1)
    nt2 = node_type.astype(jnp.int32).reshape(N, 1)
    bt2 = batch.astype(jnp.int32).reshape(N, 1)

    zeros_hbm = jnp.zeros((OUTR, HH), _f32)
    ones_hbm = jnp.ones((CH, HH), _f32)

    d0, d1 = _sc_deg(dst2d, zeros_hbm, ones_hbm)
    g0, g1, dv = _tc_init(nt2, xnum, d0, d1, W1, b1.reshape(1, H),
                          W2, b2.reshape(1, H), Wg0)
    Ws = [Wg1, Wg2, Wg3, Wg4]
    bs = [bg0, bg1, bg2, bg3]
    for i in range(4):
        s0, s1 = _sc_scatter(g0, g1, src2d, dst2d, zeros_hbm)
        g0, g1 = _tc_layer(s0, s1, g0, g1, dv, bs[i].reshape(1, H), Ws[i])
    s0, s1 = _sc_scatter(g0, g1, src2d, dst2d, zeros_hbm)
    return _tc_final(s0, s1, g0, g1, dv, bg4.reshape(1, H), bt2,
                     Wout, bout.reshape(1, 1))


# speedup vs baseline: 17.7001x; 1.6371x over previous
import functools

import jax
import jax.numpy as jnp
from jax import lax
from jax.experimental import pallas as pl
from jax.experimental.pallas import tpu as pltpu
from jax.experimental.pallas import tpu_sc as plsc

N = 100000
E = 1600000
H = 32
HH = 16
G = 64
NCLS = 8

CH = 128
SUPR = 16
NSUP = 49
RPS = SUPR * NSUP
E_PAD = 16 * RPS * CH
NROW = E_PAD // CH
OUTR = 6256
OUTR_LAST = N - 15 * OUTR
ACCR = 16 * OUTR
DSUPR = 8
DNSUP = 49
DRPW = DSUPR * DNSUP

BN = 1000
GRID = N // BN

_mesh = plsc.VectorSubcoreMesh(core_axis_name="c", subcore_axis_name="s")
_sc_params = pltpu.CompilerParams(use_tc_tiling_on_sc=False)
_f32 = jnp.float32
_out_sh = jax.ShapeDtypeStruct((N, HH), _f32)


def _zero_acc(acc, zeros_hbm, s):
    pltpu.sync_copy(zeros_hbm, acc.at[pl.ds(s * OUTR, OUTR)])


def _copy_out(acc, c, s, o0_hbm, o1_hbm):
    ob = s * OUTR

    def emit(o_hbm, nrows):
        pltpu.sync_copy(acc.at[pl.ds(ob, nrows)], o_hbm.at[pl.ds(ob, nrows)])

    for core, o_hbm in ((0, o0_hbm), (1, o1_hbm)):
        @pl.when(jnp.logical_and(c == core, s < 15))
        def _():
            emit(o_hbm, OUTR)

        @pl.when(jnp.logical_and(c == core, s == 15))
        def _():
            emit(o_hbm, OUTR_LAST)


@jax.jit
def _sc_deg(dst2d, zeros_hbm, ones_hbm):
    @functools.partial(
        pl.kernel,
        out_type=(_out_sh, _out_sh),
        mesh=_mesh,
        compiler_params=_sc_params,
        scratch_types=[
            pltpu.VMEM_SHARED((ACCR, HH), _f32),
            pltpu.VMEM((DSUPR, CH), jnp.int32),
            pltpu.VMEM((CH, HH), _f32),
            pltpu.SemaphoreType.DMA,
        ],
    )
    def k(dst_hbm, z_hbm, one_hbm, o0_hbm, o1_hbm, acc, didx, ones, ssem):
        c = lax.axis_index("c")
        s = lax.axis_index("s")
        _zero_acc(acc, z_hbm, s)
        pltpu.sync_copy(one_hbm, ones)
        plsc.subcore_barrier()

        w = s * 2 + c
        rbase = w * DRPW

        @pl.loop(0, DNSUP)
        def _(q):
            pltpu.sync_copy(dst_hbm.at[pl.ds(rbase + q * DSUPR, DSUPR)],
                            didx)
            prev = None
            for kk in range(DSUPR):
                if prev is not None:
                    prev.wait()
                prev = pltpu.async_copy(ones, acc.at[didx.at[kk]], ssem,
                                        add=True)
            prev.wait()

        plsc.subcore_barrier()
        _copy_out(acc, c, s, o0_hbm, o1_hbm)

    return k(dst2d, zeros_hbm, ones_hbm)


@jax.jit
def _sc_scatter(g0, g1, src2d, dst2d, zeros_hbm):
    @functools.partial(
        pl.kernel,
        out_type=(_out_sh, _out_sh),
        mesh=_mesh,
        compiler_params=_sc_params,
        scratch_types=[
            pltpu.VMEM_SHARED((ACCR, HH), _f32),
            pltpu.VMEM((SUPR, CH), jnp.int32),
            pltpu.VMEM((SUPR, CH), jnp.int32),
            pltpu.VMEM((4, CH, HH), _f32),
            [pltpu.SemaphoreType.DMA] * 4,
            [pltpu.SemaphoreType.DMA] * 4,
        ],
    )
    def k(g0_hbm, g1_hbm, src_hbm, dst_hbm, z_hbm, o0_hbm, o1_hbm,
          acc, sidx, didx, rows, gsems, ssems):
        c = lax.axis_index("c")
        s = lax.axis_index("s")
        _zero_acc(acc, z_hbm, s)
        plsc.subcore_barrier()

        rbase = s * RPS

        def run(g_hbm):
            @pl.loop(0, NSUP)
            def _(q):
                r0 = rbase + q * SUPR
                pltpu.sync_copy(src_hbm.at[pl.ds(r0, SUPR)], sidx)
                pltpu.sync_copy(dst_hbm.at[pl.ds(r0, SUPR)], didx)
                gd = [None] * 4
                sd = [None] * 4
                for u in range(3):
                    gd[u] = pltpu.async_copy(g_hbm.at[sidx.at[u]],
                                             rows.at[u], gsems[u])
                for kk in range(SUPR):
                    slot = kk % 4
                    slot3 = (kk + 3) % 4
                    gd[slot].wait()
                    if sd[slot3] is not None:
                        sd[slot3].wait()
                    sd[slot] = pltpu.async_copy(
                        rows.at[slot], acc.at[didx.at[kk]], ssems[slot],
                        add=True)
                    if kk + 3 < SUPR:
                        gd[slot3] = pltpu.async_copy(
                            g_hbm.at[sidx.at[kk + 3]], rows.at[slot3],
                            gsems[slot3])
                sd[(SUPR - 1) % 4].wait()

        @pl.when(c == 0)
        def _():
            run(g0_hbm)

        @pl.when(c == 1)
        def _():
            run(g1_hbm)

        plsc.subcore_barrier()
        _copy_out(acc, c, s, o0_hbm, o1_hbm)

    return k(g0, g1, src2d, dst2d, zeros_hbm)


def _full_spec(shape):
    return pl.BlockSpec(shape, lambda i: tuple(0 for _ in shape))


def _row_spec(w):
    return pl.BlockSpec((BN, w), lambda i: (i, 0))


@jax.jit
def _tc_init(nt2, xnum, d0, d1, W1, b1r, W2, b2r, Wg0):
    def body(nt_ref, xn_ref, d0_ref, d1_ref, W1_ref, b1_ref, W2_ref, b2_ref,
             Wg_ref, g0_ref, g1_ref, dv_ref):
        nt = nt_ref[...]
        oh = (nt == lax.broadcasted_iota(jnp.int32, (BN, NCLS), 1)
              ).astype(_f32)
        xt = jnp.dot(oh, W1_ref[...], preferred_element_type=_f32) + b1_ref[...]
        xn = jnp.dot(xn_ref[...], W2_ref[...], preferred_element_type=_f32) + b2_ref[...]
        deg = d0_ref[...][:, 0:1] + d1_ref[...][:, 0:1] + 1.0
        dinv = lax.rsqrt(deg)
        Wg = Wg_ref[...]
        h = (jnp.dot(xt, Wg[:H, :], preferred_element_type=_f32)
             + jnp.dot(xn, Wg[H:, :], preferred_element_type=_f32))
        g = h * dinv
        g0_ref[...] = g[:, :HH]
        g1_ref[...] = g[:, HH:]
        dv_ref[...] = dinv

    return pl.pallas_call(
        body,
        grid=(GRID,),
        in_specs=[
            _row_spec(1), _row_spec(4), _row_spec(HH), _row_spec(HH),
            _full_spec((NCLS, H)), _full_spec((1, H)),
            _full_spec((4, H)), _full_spec((1, H)),
            _full_spec((2 * H, H)),
        ],
        out_specs=[_row_spec(HH), _row_spec(HH), _row_spec(1)],
        out_shape=[
            jax.ShapeDtypeStruct((N, HH), _f32),
            jax.ShapeDtypeStruct((N, HH), _f32),
            jax.ShapeDtypeStruct((N, 1), _f32),
        ],
    )(nt2, xnum, d0, d1, W1, b1r, W2, b2r, Wg0)


@jax.jit
def _tc_layer(s0, s1, g0, g1, dv, br, W):
    def body(s0_ref, s1_ref, g0_ref, g1_ref, dv_ref, b_ref, W_ref,
             o0_ref, o1_ref):
        zl = s0_ref[...] + g0_ref[...]
        zr = s1_ref[...] + g1_ref[...]
        dinv = dv_ref[...]
        z = jnp.concatenate([zl, zr], axis=1) * dinv + b_ref[...]
        z = jnp.maximum(z, 0.0)
        h = jnp.dot(z, W_ref[...], preferred_element_type=_f32)
        g = h * dinv
        o0_ref[...] = g[:, :HH]
        o1_ref[...] = g[:, HH:]

    return pl.pallas_call(
        body,
        grid=(GRID,),
        in_specs=[
            _row_spec(HH), _row_spec(HH), _row_spec(HH), _row_spec(HH),
            _row_spec(1), _full_spec((1, H)), _full_spec((H, H)),
        ],
        out_specs=[_row_spec(HH), _row_spec(HH)],
        out_shape=[
            jax.ShapeDtypeStruct((N, HH), _f32),
            jax.ShapeDtypeStruct((N, HH), _f32),
        ],
    )(s0, s1, g0, g1, dv, br, W)


@jax.jit
def _tc_final(s0, s1, g0, g1, dv, br, bt2, Wout, boutr):
    def body(s0_ref, s1_ref, g0_ref, g1_ref, dv_ref, b_ref, bt_ref,
             Wo_ref, bo_ref, out_ref, sums, cnts):
        i = pl.program_id(0)

        @pl.when(i == 0)
        def _():
            sums[...] = jnp.zeros((G, H), _f32)
            cnts[...] = jnp.zeros((G, 1), _f32)

        zl = s0_ref[...] + g0_ref[...]
        zr = s1_ref[...] + g1_ref[...]
        z = jnp.concatenate([zl, zr], axis=1) * dv_ref[...] + b_ref[...]
        bt = bt_ref[...]
        oh = (bt == lax.broadcasted_iota(jnp.int32, (BN, G), 1)).astype(_f32)
        dn = (((0,), (0,)), ((), ()))
        sums[...] += lax.dot_general(oh, z, dn, preferred_element_type=_f32)
        cnts[...] += lax.dot_general(oh, jnp.ones((BN, 1), _f32), dn,
                                     preferred_element_type=_f32)

        @pl.when(i == GRID - 1)
        def _():
            mean = sums[...] / jnp.maximum(cnts[...], 1.0)
            out_ref[...] = (jnp.dot(mean, Wo_ref[...],
                                    preferred_element_type=_f32) + bo_ref[...])

    return pl.pallas_call(
        body,
        grid=(GRID,),
        in_specs=[
            _row_spec(HH), _row_spec(HH), _row_spec(HH), _row_spec(HH),
            _row_spec(1), _full_spec((1, H)), _row_spec(1),
            _full_spec((H, 1)), _full_spec((1, 1)),
        ],
        out_specs=pl.BlockSpec((G, 1), lambda i: (0, 0)),
        out_shape=jax.ShapeDtypeStruct((G, 1), _f32),
        scratch_shapes=[
            pltpu.VMEM((G, H), _f32),
            pltpu.VMEM((G, 1), _f32),
        ],
    )(s0, s1, g0, g1, dv, br, bt2, Wout, boutr)


def kernel(node_type, c, gm, pos, r, edge_index, batch, W1, b1, W2, b2,
           Wg0, bg0, Wg1, bg1, Wg2, bg2, Wg3, bg3, Wg4, bg4, Wout, bout):
    src = edge_index[0].astype(jnp.int32)
    dst = edge_index[1].astype(jnp.int32)
    pad = E_PAD - E
    src2d = jnp.concatenate(
        [src, jnp.zeros((pad,), jnp.int32)]).reshape(NROW, CH)
    dst2d = jnp.concatenate(
        [dst, jnp.full((pad,), N, jnp.int32)]).reshape(NROW, CH)
    xnum = jnp.stack([c, gm, pos, r], axis=-1)
    nt2 = node_type.astype(jnp.int32).reshape(N, 1)
    bt2 = batch.astype(jnp.int32).reshape(N, 1)

    zeros_hbm = jnp.zeros((OUTR, HH), _f32)
    ones_hbm = jnp.ones((CH, HH), _f32)

    d0, d1 = _sc_deg(dst2d, zeros_hbm, ones_hbm)
    g0, g1, dv = _tc_init(nt2, xnum, d0, d1, W1, b1.reshape(1, H),
                          W2, b2.reshape(1, H), Wg0)
    Ws = [Wg1, Wg2, Wg3, Wg4]
    bs = [bg0, bg1, bg2, bg3]
    for i in range(4):
        s0, s1 = _sc_scatter(g0, g1, src2d, dst2d, zeros_hbm)
        g0, g1 = _tc_layer(s0, s1, g0, g1, dv, bs[i].reshape(1, H), Ws[i])
    s0, s1 = _sc_scatter(g0, g1, src2d, dst2d, zeros_hbm)
    return _tc_final(s0, s1, g0, g1, dv, bg4.reshape(1, H), bt2,
                     Wout, bout.reshape(1, 1))


# speedup vs baseline: 23.1250x; 1.3065x over previous
import functools

import jax
import jax.numpy as jnp
from jax import lax
from jax.experimental import pallas as pl
from jax.experimental.pallas import tpu as pltpu
from jax.experimental.pallas import tpu_sc as plsc

N = 100000
E = 1600000
H = 32
HH = 16
G = 64
NCLS = 8

CH = 128
SUPR = 56
NSUP = 14
RING = 6
RPS = SUPR * NSUP
E_PAD = 16 * RPS * CH
NROW = E_PAD // CH
OUTR = 6256
OUTR_LAST = N - 15 * OUTR
ACCR = 16 * OUTR
DSUPR = 56
DNSUP = 7
DRPW = DSUPR * DNSUP
DRING = 4

BN = 4000
GRID = N // BN

_mesh = plsc.VectorSubcoreMesh(core_axis_name="c", subcore_axis_name="s")
_sc_params = pltpu.CompilerParams(use_tc_tiling_on_sc=False)
_f32 = jnp.float32
_out_sh = jax.ShapeDtypeStruct((N, HH), _f32)


def _zero_acc(acc, zeros_hbm, s):
    pltpu.sync_copy(zeros_hbm, acc.at[pl.ds(s * OUTR, OUTR)])


def _copy_out(acc, c, s, o0_hbm, o1_hbm):
    ob = s * OUTR

    def emit(o_hbm, nrows):
        pltpu.sync_copy(acc.at[pl.ds(ob, nrows)], o_hbm.at[pl.ds(ob, nrows)])

    for core, o_hbm in ((0, o0_hbm), (1, o1_hbm)):
        @pl.when(jnp.logical_and(c == core, s < 15))
        def _():
            emit(o_hbm, OUTR)

        @pl.when(jnp.logical_and(c == core, s == 15))
        def _():
            emit(o_hbm, OUTR_LAST)


@jax.jit
def _sc_deg(dst2d, zeros_hbm, ones_hbm):
    @functools.partial(
        pl.kernel,
        out_type=(_out_sh, _out_sh),
        mesh=_mesh,
        compiler_params=_sc_params,
        scratch_types=[
            pltpu.VMEM_SHARED((ACCR, HH), _f32),
            pltpu.VMEM((DSUPR, CH), jnp.int32),
            pltpu.VMEM((CH, HH), _f32),
            [pltpu.SemaphoreType.DMA] * DRING,
        ],
    )
    def k(dst_hbm, z_hbm, one_hbm, o0_hbm, o1_hbm, acc, didx, ones, ssems):
        c = lax.axis_index("c")
        s = lax.axis_index("s")
        _zero_acc(acc, z_hbm, s)
        pltpu.sync_copy(one_hbm, ones)
        plsc.subcore_barrier()

        w = s * 2 + c
        rbase = w * DRPW

        @pl.loop(0, DNSUP)
        def _(q):
            pltpu.sync_copy(dst_hbm.at[pl.ds(rbase + q * DSUPR, DSUPR)],
                            didx)
            sd = [None] * DRING
            for kk in range(DSUPR):
                slot = kk % DRING
                if sd[slot] is not None:
                    sd[slot].wait()
                sd[slot] = pltpu.async_copy(ones, acc.at[didx.at[kk]],
                                            ssems[slot], add=True)
            for slot in range(DRING):
                if sd[slot] is not None:
                    sd[slot].wait()

        plsc.subcore_barrier()
        _copy_out(acc, c, s, o0_hbm, o1_hbm)

    return k(dst2d, zeros_hbm, ones_hbm)


@jax.jit
def _sc_scatter(g0, g1, ed2d, zeros_hbm):
    @functools.partial(
        pl.kernel,
        out_type=(_out_sh, _out_sh),
        mesh=_mesh,
        compiler_params=_sc_params,
        scratch_types=[
            pltpu.VMEM_SHARED((ACCR, HH), _f32),
            pltpu.VMEM((2 * SUPR, CH), jnp.int32),
            pltpu.VMEM((RING, CH, HH), _f32),
            [pltpu.SemaphoreType.DMA] * RING,
            [pltpu.SemaphoreType.DMA] * RING,
        ],
    )
    def k(g0_hbm, g1_hbm, ed_hbm, z_hbm, o0_hbm, o1_hbm,
          acc, eidx, rows, gsems, ssems):
        c = lax.axis_index("c")
        s = lax.axis_index("s")
        _zero_acc(acc, z_hbm, s)
        plsc.subcore_barrier()

        rbase = 2 * s * RPS

        def run(g_hbm):
            @pl.loop(0, NSUP)
            def _(q):
                r0 = rbase + q * 2 * SUPR
                pltpu.sync_copy(ed_hbm.at[pl.ds(r0, 2 * SUPR)], eidx)
                gd = [None] * RING
                sd = [None] * RING
                for u in range(RING - 1):
                    gd[u] = pltpu.async_copy(g_hbm.at[eidx.at[2 * u]],
                                             rows.at[u], gsems[u])
                for kk in range(SUPR):
                    slot = kk % RING
                    pslot = (kk - 1) % RING
                    gd[slot].wait()
                    sd[slot] = pltpu.async_copy(
                        rows.at[slot], acc.at[eidx.at[2 * kk + 1]],
                        ssems[slot], add=True)
                    nxt = kk + RING - 1
                    if nxt < SUPR:
                        if sd[pslot] is not None:
                            sd[pslot].wait()
                        gd[pslot] = pltpu.async_copy(
                            g_hbm.at[eidx.at[2 * nxt]], rows.at[pslot],
                            gsems[pslot])
                for i in range(SUPR - RING, SUPR):
                    sd[i % RING].wait()

        @pl.when(c == 0)
        def _():
            run(g0_hbm)

        @pl.when(c == 1)
        def _():
            run(g1_hbm)

        plsc.subcore_barrier()
        _copy_out(acc, c, s, o0_hbm, o1_hbm)

    return k(g0, g1, ed2d, zeros_hbm)


def _full_spec(shape):
    return pl.BlockSpec(shape, lambda i: tuple(0 for _ in shape))


def _row_spec(w):
    return pl.BlockSpec((BN, w), lambda i: (i, 0))


@jax.jit
def _tc_init(nt2, xnum, d0, d1, W1, b1r, W2, b2r, Wg0):
    def body(nt_ref, xn_ref, d0_ref, d1_ref, W1_ref, b1_ref, W2_ref, b2_ref,
             Wg_ref, g0_ref, g1_ref, dv_ref):
        nt = nt_ref[...]
        oh = (nt == lax.broadcasted_iota(jnp.int32, (BN, NCLS), 1)
              ).astype(_f32)
        xt = jnp.dot(oh, W1_ref[...], preferred_element_type=_f32) + b1_ref[...]
        xn = jnp.dot(xn_ref[...], W2_ref[...], preferred_element_type=_f32) + b2_ref[...]
        deg = d0_ref[...][:, 0:1] + d1_ref[...][:, 0:1] + 1.0
        dinv = lax.rsqrt(deg)
        Wg = Wg_ref[...]
        h = (jnp.dot(xt, Wg[:H, :], preferred_element_type=_f32)
             + jnp.dot(xn, Wg[H:, :], preferred_element_type=_f32))
        g = h * dinv
        g0_ref[...] = g[:, :HH]
        g1_ref[...] = g[:, HH:]
        dv_ref[...] = dinv

    return pl.pallas_call(
        body,
        grid=(GRID,),
        in_specs=[
            _row_spec(1), _row_spec(4), _row_spec(HH), _row_spec(HH),
            _full_spec((NCLS, H)), _full_spec((1, H)),
            _full_spec((4, H)), _full_spec((1, H)),
            _full_spec((2 * H, H)),
        ],
        out_specs=[_row_spec(HH), _row_spec(HH), _row_spec(1)],
        out_shape=[
            jax.ShapeDtypeStruct((N, HH), _f32),
            jax.ShapeDtypeStruct((N, HH), _f32),
            jax.ShapeDtypeStruct((N, 1), _f32),
        ],
    )(nt2, xnum, d0, d1, W1, b1r, W2, b2r, Wg0)


@jax.jit
def _tc_layer(s0, s1, g0, g1, dv, br, W):
    def body(s0_ref, s1_ref, g0_ref, g1_ref, dv_ref, b_ref, W_ref,
             o0_ref, o1_ref):
        zl = s0_ref[...] + g0_ref[...]
        zr = s1_ref[...] + g1_ref[...]
        dinv = dv_ref[...]
        z = jnp.concatenate([zl, zr], axis=1) * dinv + b_ref[...]
        z = jnp.maximum(z, 0.0)
        h = jnp.dot(z, W_ref[...], preferred_element_type=_f32)
        g = h * dinv
        o0_ref[...] = g[:, :HH]
        o1_ref[...] = g[:, HH:]

    return pl.pallas_call(
        body,
        grid=(GRID,),
        in_specs=[
            _row_spec(HH), _row_spec(HH), _row_spec(HH), _row_spec(HH),
            _row_spec(1), _full_spec((1, H)), _full_spec((H, H)),
        ],
        out_specs=[_row_spec(HH), _row_spec(HH)],
        out_shape=[
            jax.ShapeDtypeStruct((N, HH), _f32),
            jax.ShapeDtypeStruct((N, HH), _f32),
        ],
    )(s0, s1, g0, g1, dv, br, W)


@jax.jit
def _tc_final(s0, s1, g0, g1, dv, br, bt2, Wout, boutr):
    def body(s0_ref, s1_ref, g0_ref, g1_ref, dv_ref, b_ref, bt_ref,
             Wo_ref, bo_ref, out_ref, sums, cnts):
        i = pl.program_id(0)

        @pl.when(i == 0)
        def _():
            sums[...] = jnp.zeros((G, H), _f32)
            cnts[...] = jnp.zeros((G, 1), _f32)

        zl = s0_ref[...] + g0_ref[...]
        zr = s1_ref[...] + g1_ref[...]
        z = jnp.concatenate([zl, zr], axis=1) * dv_ref[...] + b_ref[...]
        bt = bt_ref[...]
        oh = (bt == lax.broadcasted_iota(jnp.int32, (BN, G), 1)).astype(_f32)
        dn = (((0,), (0,)), ((), ()))
        sums[...] += lax.dot_general(oh, z, dn, preferred_element_type=_f32)
        cnts[...] += lax.dot_general(oh, jnp.ones((BN, 1), _f32), dn,
                                     preferred_element_type=_f32)

        @pl.when(i == GRID - 1)
        def _():
            mean = sums[...] / jnp.maximum(cnts[...], 1.0)
            out_ref[...] = (jnp.dot(mean, Wo_ref[...],
                                    preferred_element_type=_f32) + bo_ref[...])

    return pl.pallas_call(
        body,
        grid=(GRID,),
        in_specs=[
            _row_spec(HH), _row_spec(HH), _row_spec(HH), _row_spec(HH),
            _row_spec(1), _full_spec((1, H)), _row_spec(1),
            _full_spec((H, 1)), _full_spec((1, 1)),
        ],
        out_specs=pl.BlockSpec((G, 1), lambda i: (0, 0)),
        out_shape=jax.ShapeDtypeStruct((G, 1), _f32),
        scratch_shapes=[
            pltpu.VMEM((G, H), _f32),
            pltpu.VMEM((G, 1), _f32),
        ],
    )(s0, s1, g0, g1, dv, br, bt2, Wout, boutr)


def kernel(node_type, c, gm, pos, r, edge_index, batch, W1, b1, W2, b2,
           Wg0, bg0, Wg1, bg1, Wg2, bg2, Wg3, bg3, Wg4, bg4, Wout, bout):
    src = edge_index[0].astype(jnp.int32)
    dst = edge_index[1].astype(jnp.int32)
    pad = E_PAD - E
    src2d = jnp.concatenate(
        [src, jnp.zeros((pad,), jnp.int32)]).reshape(NROW, CH)
    dst2d = jnp.concatenate(
        [dst, jnp.full((pad,), N, jnp.int32)]).reshape(NROW, CH)
    ed2d = jnp.stack([src2d, dst2d], axis=1).reshape(2 * NROW, CH)
    xnum = jnp.stack([c, gm, pos, r], axis=-1)
    nt2 = node_type.astype(jnp.int32).reshape(N, 1)
    bt2 = batch.astype(jnp.int32).reshape(N, 1)

    zeros_hbm = jnp.zeros((OUTR, HH), _f32)
    ones_hbm = jnp.ones((CH, HH), _f32)

    d0, d1 = _sc_deg(dst2d, zeros_hbm, ones_hbm)
    g0, g1, dv = _tc_init(nt2, xnum, d0, d1, W1, b1.reshape(1, H),
                          W2, b2.reshape(1, H), Wg0)
    Ws = [Wg1, Wg2, Wg3, Wg4]
    bs = [bg0, bg1, bg2, bg3]
    for i in range(4):
        s0, s1 = _sc_scatter(g0, g1, ed2d, zeros_hbm)
        g0, g1 = _tc_layer(s0, s1, g0, g1, dv, bs[i].reshape(1, H), Ws[i])
    s0, s1 = _sc_scatter(g0, g1, ed2d, zeros_hbm)
    return _tc_final(s0, s1, g0, g1, dv, bg4.reshape(1, H), bt2,
                     Wout, bout.reshape(1, 1))
